# Initial kernel scaffold; baseline (speedup 1.0000x reference)
import jax
import jax.numpy as jnp
from jax.experimental import pallas as pl


def kernel(c3_conv1_w, c3_conv1_b, c3_conv2_w, c3_conv2_b, c4_conv1_w, c4_conv1_b, c4_conv2_w, c4_conv2_b, c5_conv1_w, c5_conv1_b, c5_conv2_w, c5_conv2_b, c5_conv3_w, c5_conv3_b, c5_conv4_w, c5_conv4_b, c3, c4, c5, rh45, rw45, rh34, rw34):
    raise NotImplementedError("write your pallas kernel here")


# speedup vs baseline: 2.6518x; 2.6518x over previous
import jax
import jax.numpy as jnp
from jax.experimental import pallas as pl
from jax.experimental.pallas import tpu as pltpu

OUT = 256
_VMEM = 56 * 1024 * 1024
_BF = jnp.bfloat16
_F32 = jnp.float32


def _cp(*sem):
    return pltpu.CompilerParams(dimension_semantics=sem,
                                vmem_limit_bytes=_VMEM)


def _pad1(x):
    h, w, c = x.shape
    zc = jnp.zeros((h, 1, c), x.dtype)
    x = jnp.concatenate([zc, x, zc], axis=1)
    zr = jnp.zeros((1, w + 2, c), x.dtype)
    return jnp.concatenate([zr, x, zr], axis=0)


def _conv3x3_write(xp, w2col, b, o_ref, th):
    hp, wp, c = xp.shape
    h, w = hp - 2, wp - 2
    cols = [xp[:, dx:dx + w, :] for dx in range(3)]
    for t in range(0, h, th):
        patches = jnp.concatenate(
            [cols[dx][t + dy:t + dy + th].reshape(th * w, c)
             for dy in range(3) for dx in range(3)], axis=1)
        acc = jnp.dot(patches, w2col,
                      preferred_element_type=_F32) + b
        o_ref[0, t:t + th] = acc.reshape(th, w, OUT).astype(o_ref.dtype)


def _upsample(x, rh, rw):
    ho = rh.shape[0]
    wo, wi = rw.shape
    y = jnp.einsum("oh,hwc->owc", rh, x,
                   preferred_element_type=_F32)
    rwb = jnp.broadcast_to(rw, (ho, wo, wi))
    return jnp.einsum("row,rwc->roc", rwb, y.astype(_BF),
                      preferred_element_type=_F32)


def _k_a(c5_ref, w51_ref, b51_ref, w52_ref, b52_ref, w6_ref, b6_ref,
         w7_ref, b7_ref, c5c_ref, p5_ref, p6_ref, p7_ref):
    xb = c5_ref[0].astype(_BF)
    cin = xb.shape[-1]
    w51 = w51_ref[...].astype(_BF)
    t = jnp.dot(xb.reshape(256, cin), w51,
                preferred_element_type=_F32) + b51_ref[...]
    c5c_ref[0] = t.reshape(16, 16, OUT).astype(_BF)
    xp = _pad1(t.astype(_BF).reshape(16, 16, OUT))
    _conv3x3_write(xp, w52_ref[...].astype(_BF), b52_ref[...], p5_ref, 16)
    e = xb.reshape(8, 2, 16, cin)[:, 0]
    e = e.reshape(8, 8, 2, cin)[:, :, 0]
    w6 = w6_ref[...].astype(_BF)
    p6 = jnp.dot(e.reshape(64, cin), w6,
                 preferred_element_type=_F32) + b6_ref[...]
    p6_ref[0] = p6.reshape(8, 8, OUT)
    e7 = e.reshape(4, 2, 8, cin)[:, 0]
    e7 = e7.reshape(4, 4, 2, cin)[:, :, 0]
    t7 = jnp.dot(e7.reshape(16, cin), w6,
                 preferred_element_type=_F32) + b6_ref[...]
    p7 = jnp.dot(t7.astype(_BF), w7_ref[...].astype(_BF),
                 preferred_element_type=_F32) + b7_ref[...]
    p7_ref[0] = p7.reshape(4, 4, OUT)


def _k_b(c5c_ref, c4_ref, rh_ref, rw_ref, w41_ref, b41_ref, w42_ref,
         b42_ref, s4_ref, p4_ref):
    up = _upsample(c5c_ref[0], rh_ref[...].astype(_BF),
                   rw_ref[...].astype(_BF))
    c4b = c4_ref[0].astype(_BF)
    cin = c4b.shape[-1]
    lat = jnp.dot(c4b.reshape(1024, cin), w41_ref[...].astype(_BF),
                  preferred_element_type=_F32) + b41_ref[...]
    s4 = up + lat.reshape(32, 32, OUT)
    s4b = s4.astype(_BF)
    s4_ref[0] = s4b
    _conv3x3_write(_pad1(s4b), w42_ref[...].astype(_BF), b42_ref[...],
                   p4_ref, 32)


def _k_c(s4_ref, c3_ref, rh_ref, rw_ref, w31_ref, b31_ref, w32_ref,
         b32_ref, p3_ref):
    up = _upsample(s4_ref[0], rh_ref[...].astype(_BF),
                   rw_ref[...].astype(_BF))
    c3b = c3_ref[0].astype(_BF)
    cin = c3b.shape[-1]
    lat = jnp.dot(c3b.reshape(4096, cin), w31_ref[...].astype(_BF),
                  preferred_element_type=_F32) + b31_ref[...]
    s3b = (up + lat.reshape(64, 64, OUT)).astype(_BF)
    _conv3x3_write(_pad1(s3b), w32_ref[...].astype(_BF), b32_ref[...],
                   p3_ref, 16)


def _full(shape):
    nd = len(shape)
    return pl.BlockSpec(shape, lambda i, nd=nd: (0,) * nd)


def _batched(shape):
    nd = len(shape)
    return pl.BlockSpec((1,) + shape, lambda i, nd=nd: (i,) + (0,) * nd)


def kernel(c3_conv1_w, c3_conv1_b, c3_conv2_w, c3_conv2_b,
           c4_conv1_w, c4_conv1_b, c4_conv2_w, c4_conv2_b,
           c5_conv1_w, c5_conv1_b, c5_conv2_w, c5_conv2_b,
           c5_conv3_w, c5_conv3_b, c5_conv4_w, c5_conv4_b,
           c3, c4, c5, rh45, rw45, rh34, rw34):
    n = c5.shape[0]
    c3c, c4c, c5c_in = c3.shape[-1], c4.shape[-1], c5.shape[-1]
    w52 = c5_conv2_w.reshape(9 * OUT, OUT)
    w42 = c4_conv2_w.reshape(9 * OUT, OUT)
    w32 = c3_conv2_w.reshape(9 * OUT, OUT)
    b = lambda v: v.reshape(1, OUT)

    c5c, p5, p6, p7 = pl.pallas_call(
        _k_a,
        grid=(n,),
        in_specs=[
            _batched((16, 16, c5c_in)),
            _full((c5c_in, OUT)), _full((1, OUT)),
            _full((9 * OUT, OUT)), _full((1, OUT)),
            _full((c5c_in, OUT)), _full((1, OUT)),
            _full((OUT, OUT)), _full((1, OUT)),
        ],
        out_specs=[
            _batched((16, 16, OUT)), _batched((16, 16, OUT)),
            _batched((8, 8, OUT)), _batched((4, 4, OUT)),
        ],
        out_shape=[
            jax.ShapeDtypeStruct((n, 16, 16, OUT), _BF),
            jax.ShapeDtypeStruct((n, 16, 16, OUT), _F32),
            jax.ShapeDtypeStruct((n, 8, 8, OUT), _F32),
            jax.ShapeDtypeStruct((n, 4, 4, OUT), _F32),
        ],
        compiler_params=_cp("parallel"),
    )(c5, c5_conv1_w, b(c5_conv1_b), w52, b(c5_conv2_b),
      c5_conv3_w, b(c5_conv3_b), c5_conv4_w, b(c5_conv4_b))

    s4, p4 = pl.pallas_call(
        _k_b,
        grid=(n,),
        in_specs=[
            _batched((16, 16, OUT)),
            _batched((32, 32, c4c)),
            _full((32, 16)), _full((32, 16)),
            _full((c4c, OUT)), _full((1, OUT)),
            _full((9 * OUT, OUT)), _full((1, OUT)),
        ],
        out_specs=[_batched((32, 32, OUT)), _batched((32, 32, OUT))],
        out_shape=[
            jax.ShapeDtypeStruct((n, 32, 32, OUT), _BF),
            jax.ShapeDtypeStruct((n, 32, 32, OUT), _F32),
        ],
        compiler_params=_cp("parallel"),
    )(c5c, c4, rh45, rw45, c4_conv1_w, b(c4_conv1_b), w42, b(c4_conv2_b))

    p3 = pl.pallas_call(
        _k_c,
        grid=(n,),
        in_specs=[
            _batched((32, 32, OUT)),
            _batched((64, 64, c3c)),
            _full((64, 32)), _full((64, 32)),
            _full((c3c, OUT)), _full((1, OUT)),
            _full((9 * OUT, OUT)), _full((1, OUT)),
        ],
        out_specs=_batched((64, 64, OUT)),
        out_shape=jax.ShapeDtypeStruct((n, 64, 64, OUT), _F32),
        compiler_params=_cp("parallel"),
    )(s4, c3, rh34, rw34, c3_conv1_w, b(c3_conv1_b), w32, b(c3_conv2_b))

    return p3, p4, p5, p6, p7


# speedup vs baseline: 2.9306x; 1.1051x over previous
import jax
import jax.numpy as jnp
from jax.experimental import pallas as pl
from jax.experimental.pallas import tpu as pltpu

OUT = 256
_VMEM = 60 * 1024 * 1024
_BF = jnp.bfloat16
_F32 = jnp.float32


def _cp(*sem):
    return pltpu.CompilerParams(dimension_semantics=sem,
                                vmem_limit_bytes=_VMEM)


def _padw(x):
    h, w, c = x.shape
    zc = jnp.zeros((h, 1, c), x.dtype)
    return jnp.concatenate([zc, x, zc], axis=1)


def _pad1(x):
    h, w, c = x.shape
    xp = _padw(x)
    zr = jnp.zeros((1, w + 2, c), x.dtype)
    return jnp.concatenate([zr, xp, zr], axis=0)


def _conv3x3_write(xp, w2col, b, o_ref, th, row0=0):
    hp, wp, c = xp.shape
    h, w = hp - 2, wp - 2
    cols = [xp[:, dx:dx + w, :] for dx in range(3)]
    for t in range(0, h, th):
        patches = jnp.concatenate(
            [cols[dx][t + dy:t + dy + th].reshape(th * w, c)
             for dy in range(3) for dx in range(3)], axis=1)
        acc = jnp.dot(patches, w2col,
                      preferred_element_type=_F32) + b
        o_ref[0, row0 + t:row0 + t + th] = (
            acc.reshape(th, w, OUT).astype(o_ref.dtype))


def _upsample(x, rh, rw):
    ho = rh.shape[0]
    wo, wi = rw.shape
    y = jnp.einsum("oh,hwc->owc", rh, x,
                   preferred_element_type=_F32)
    rwb = jnp.broadcast_to(rw, (ho, wo, wi))
    return jnp.einsum("row,rwc->roc", rwb, y.astype(_BF),
                      preferred_element_type=_F32)


def _wup(y, rw):
    h = y.shape[0]
    wo, wi = rw.shape
    rwb = jnp.broadcast_to(rw, (h, wo, wi))
    return jnp.einsum("row,rwc->roc", rwb, y,
                      preferred_element_type=_F32)


def _k(c5_ref, c4_ref, c3h_ref, c3t_ref, c3b_ref,
       rh45_ref, rw45_ref, rh34_ref, rw34_ref,
       w51_ref, b51_ref, w52_ref, b52_ref, w6_ref, b6_ref, w7_ref, b7_ref,
       w41_ref, b41_ref, w42_ref, b42_ref, w31_ref, b31_ref, w32_ref,
       b32_ref, p3_ref, p4_ref, p5_ref, p6_ref, p7_ref, s4_scr):
    g = pl.program_id(1)

    @pl.when(g == 0)
    def _stage_ab():
        xb = c5_ref[0].astype(_BF)
        cin = xb.shape[-1]
        w51 = w51_ref[...].astype(_BF)
        t = jnp.dot(xb.reshape(256, cin), w51,
                    preferred_element_type=_F32) + b51_ref[...]
        tb = t.astype(_BF).reshape(16, 16, OUT)
        _conv3x3_write(_pad1(tb), w52_ref[...].astype(_BF), b52_ref[...],
                       p5_ref, 16)
        e = xb.reshape(8, 2, 16, cin)[:, 0]
        e = e.reshape(8, 8, 2, cin)[:, :, 0]
        w6 = w6_ref[...].astype(_BF)
        p6 = jnp.dot(e.reshape(64, cin), w6,
                     preferred_element_type=_F32) + b6_ref[...]
        p6_ref[0] = p6.reshape(8, 8, OUT)
        e7 = e.reshape(4, 2, 8, cin)[:, 0]
        e7 = e7.reshape(4, 4, 2, cin)[:, :, 0]
        t7 = jnp.dot(e7.reshape(16, cin), w6,
                     preferred_element_type=_F32) + b6_ref[...]
        p7 = jnp.dot(t7.astype(_BF), w7_ref[...].astype(_BF),
                     preferred_element_type=_F32) + b7_ref[...]
        p7_ref[0] = p7.reshape(4, 4, OUT)
        up = _upsample(tb, rh45_ref[...].astype(_BF),
                       rw45_ref[...].astype(_BF))
        c4b = c4_ref[0].astype(_BF)
        c4c = c4b.shape[-1]
        lat = jnp.dot(c4b.reshape(1024, c4c), w41_ref[...].astype(_BF),
                      preferred_element_type=_F32) + b41_ref[...]
        s4b = (up + lat.reshape(32, 32, OUT)).astype(_BF)
        s4_scr[...] = s4b
        _conv3x3_write(_pad1(s4b), w42_ref[...].astype(_BF),
                       b42_ref[...], p4_ref, 16)

    s4b = s4_scr[...]
    y64 = jnp.einsum("oh,hwc->owc", rh34_ref[...].astype(_BF), s4b,
                     preferred_element_type=_F32)
    w31 = w31_ref[...].astype(_BF)
    w32 = w32_ref[...].astype(_BF)
    rw34 = rw34_ref[...].astype(_BF)
    c3c = c3h_ref.shape[-1]
    zrow = jnp.zeros((1, 64, OUT), _BF)

    def _stage_c(gs):
        if gs == 0:
            y33 = y64[0:33]
            c3rows = jnp.concatenate([c3h_ref[0], c3b_ref[0]], axis=0)
        else:
            y33 = y64[31:64]
            c3rows = jnp.concatenate([c3t_ref[0], c3h_ref[0]], axis=0)
        up33 = _wup(y33.astype(_BF), rw34)
        lat = jnp.dot(c3rows.astype(_BF).reshape(33 * 64, c3c), w31,
                      preferred_element_type=_F32) + b31_ref[...]
        s3 = (up33 + lat.reshape(33, 64, OUT)).astype(_BF)
        if gs == 0:
            s3pad = jnp.concatenate([zrow, s3], axis=0)
        else:
            s3pad = jnp.concatenate([s3, zrow], axis=0)
        _conv3x3_write(_padw(s3pad), w32, b32_ref[...], p3_ref, 16)

    @pl.when(g == 0)
    def _c0():
        _stage_c(0)

    @pl.when(g == 1)
    def _c1():
        _stage_c(1)


def _full(shape):
    nd = len(shape)
    return pl.BlockSpec(shape, lambda b, g, nd=nd: (0,) * nd)


def kernel(c3_conv1_w, c3_conv1_b, c3_conv2_w, c3_conv2_b,
           c4_conv1_w, c4_conv1_b, c4_conv2_w, c4_conv2_b,
           c5_conv1_w, c5_conv1_b, c5_conv2_w, c5_conv2_b,
           c5_conv3_w, c5_conv3_b, c5_conv4_w, c5_conv4_b,
           c3, c4, c5, rh45, rw45, rh34, rw34):
    n = c5.shape[0]
    c3c, c4c, c5c_in = c3.shape[-1], c4.shape[-1], c5.shape[-1]
    w52 = c5_conv2_w.reshape(9 * OUT, OUT)
    w42 = c4_conv2_w.reshape(9 * OUT, OUT)
    w32 = c3_conv2_w.reshape(9 * OUT, OUT)
    b = lambda v: v.reshape(1, OUT)

    p3, p4, p5, p6, p7 = pl.pallas_call(
        _k,
        grid=(n, 2),
        in_specs=[
            pl.BlockSpec((1, 16, 16, c5c_in), lambda b_, g: (b_, 0, 0, 0)),
            pl.BlockSpec((1, 32, 32, c4c), lambda b_, g: (b_, 0, 0, 0)),
            pl.BlockSpec((1, 32, 64, c3c), lambda b_, g: (b_, g, 0, 0)),
            pl.BlockSpec((1, 1, 64, c3c), lambda b_, g: (b_, 31 * g, 0, 0)),
            pl.BlockSpec((1, 1, 64, c3c),
                         lambda b_, g: (b_, 32 + 31 * g, 0, 0)),
            _full((32, 16)), _full((32, 16)),
            _full((64, 32)), _full((64, 32)),
            _full((c5c_in, OUT)), _full((1, OUT)),
            _full((9 * OUT, OUT)), _full((1, OUT)),
            _full((c5c_in, OUT)), _full((1, OUT)),
            _full((OUT, OUT)), _full((1, OUT)),
            _full((c4c, OUT)), _full((1, OUT)),
            _full((9 * OUT, OUT)), _full((1, OUT)),
            _full((c3c, OUT)), _full((1, OUT)),
            _full((9 * OUT, OUT)), _full((1, OUT)),
        ],
        out_specs=[
            pl.BlockSpec((1, 32, 64, OUT), lambda b_, g: (b_, g, 0, 0)),
            pl.BlockSpec((1, 32, 32, OUT), lambda b_, g: (b_, 0, 0, 0)),
            pl.BlockSpec((1, 16, 16, OUT), lambda b_, g: (b_, 0, 0, 0)),
            pl.BlockSpec((1, 8, 8, OUT), lambda b_, g: (b_, 0, 0, 0)),
            pl.BlockSpec((1, 4, 4, OUT), lambda b_, g: (b_, 0, 0, 0)),
        ],
        out_shape=[
            jax.ShapeDtypeStruct((n, 64, 64, OUT), _F32),
            jax.ShapeDtypeStruct((n, 32, 32, OUT), _F32),
            jax.ShapeDtypeStruct((n, 16, 16, OUT), _F32),
            jax.ShapeDtypeStruct((n, 8, 8, OUT), _F32),
            jax.ShapeDtypeStruct((n, 4, 4, OUT), _F32),
        ],
        scratch_shapes=[pltpu.VMEM((32, 32, OUT), _BF)],
        compiler_params=_cp("parallel", "arbitrary"),
    )(c5, c4, c3, c3, c3, rh45, rw45, rh34, rw34,
      c5_conv1_w, b(c5_conv1_b), w52, b(c5_conv2_b),
      c5_conv3_w, b(c5_conv3_b), c5_conv4_w, b(c5_conv4_b),
      c4_conv1_w, b(c4_conv1_b), w42, b(c4_conv2_b),
      c3_conv1_w, b(c3_conv1_b), w32, b(c3_conv2_b))
    return p3, p4, p5, p6, p7
